# baseline (speedup 1.0000x reference)
import functools

import jax
import jax.numpy as jnp
from jax import lax
from jax.experimental import pallas as pl
from jax.experimental.pallas import tpu as pltpu
from jax.experimental.pallas import tpu_sc as plsc

D_MODEL = 64
NUM_CORES = 2
NUM_SUBCORES = 16
NUM_WORKERS = NUM_CORES * NUM_SUBCORES
GROUP = 128
NBUF = 4
LANES = 16


@functools.lru_cache(maxsize=None)
def _build(n_t: int, n_a: int):
    a_tiles = n_a // GROUP
    f_tiles = D_MODEL // 8

    mesh = plsc.VectorSubcoreMesh(
        core_axis_name="c",
        subcore_axis_name="s",
        num_cores=NUM_CORES,
        num_subcores=NUM_SUBCORES,
    )

    @functools.partial(
        pl.kernel,
        out_type=jax.ShapeDtypeStruct(
            (n_t, f_tiles, a_tiles, 8, GROUP), jnp.float32
        ),
        mesh=mesh,
        scratch_types=[
            pltpu.VMEM((n_t, GROUP), jnp.int32),
            pltpu.VMEM((NBUF, GROUP, D_MODEL), jnp.float32),
            pltpu.VMEM((NBUF, f_tiles, 8, GROUP), jnp.float32),
            pltpu.SemaphoreType.DMA((NBUF,)),
            pltpu.SemaphoreType.DMA((NBUF,)),
        ],
        compiler_params=pltpu.CompilerParams(
            use_tc_tiling_on_sc=False, needs_layout_passes=False
        ),
    )
    def emb_kernel(xt_hbm, table_hbm, out_hbm, idx_v, rows_v, tbuf_v, sem_g, sem_w):
        wid = lax.axis_index("s") * NUM_CORES + lax.axis_index("c")

        pltpu.sync_copy(
            xt_hbm.at[pl.ds(0, n_t), pl.ds(wid * GROUP, GROUP)], idx_v
        )

        def fire_gather(t, buf):
            pltpu.async_copy(
                table_hbm.at[idx_v.at[t]], rows_v.at[buf], sem_g.at[buf]
            )

        def wait_gather(buf):
            pltpu.make_async_copy(
                table_hbm.at[idx_v.at[0]], rows_v.at[buf], sem_g.at[buf]
            ).wait()

        def fire_write(t, buf):
            pltpu.async_copy(
                tbuf_v.at[buf], out_hbm.at[t, pl.ds(0, f_tiles), wid],
                sem_w.at[buf],
            )

        def wait_write(buf):
            pltpu.make_async_copy(
                tbuf_v.at[buf], out_hbm.at[0, pl.ds(0, f_tiles), 0],
                sem_w.at[buf],
            ).wait()

        for b in range(NBUF):
            fire_gather(b, b)

        base = lax.iota(jnp.int32, LANES)
        cvecs = [base + (c0 * LANES) for c0 in range(GROUP // LANES)]

        @pl.loop(0, n_t // NBUF)
        def _outer(ti):
            for b in range(NBUF):
                t = ti * NBUF + b
                wait_gather(b)

                @pl.when(t >= NBUF)
                def _():
                    wait_write(b)

                @pl.loop(0, f_tiles)
                def _feat(fi):
                    for r in range(8):
                        fvec = jnp.full((LANES,), fi * 8 + r, jnp.int32)
                        for c0 in range(GROUP // LANES):
                            v = plsc.load_gather(
                                rows_v.at[b], [cvecs[c0], fvec]
                            )
                            tbuf_v[b, fi, r, pl.ds(c0 * LANES, LANES)] = v * 8.0

                fire_write(t, b)

                @pl.when(t + NBUF < n_t)
                def _():
                    fire_gather(t + NBUF, b)

        for b in range(NBUF):
            wait_write(b)

    return emb_kernel


def kernel(x, table):
    n_a, n_t = x.shape
    xt = jnp.transpose(x).astype(jnp.int32)
    out5 = _build(n_t, n_a)(xt, table)
    return out5.transpose(2, 4, 0, 1, 3).reshape(n_a, n_t, D_MODEL)


# speedup vs baseline: 1.7436x; 1.7436x over previous
import functools

import jax
import jax.numpy as jnp
from jax import lax
from jax.experimental import pallas as pl
from jax.experimental.pallas import tpu as pltpu
from jax.experimental.pallas import tpu_sc as plsc

D_MODEL = 64
NUM_CORES = 2
NUM_SUBCORES = 16
NUM_WORKERS = NUM_CORES * NUM_SUBCORES
GROUP = 128
NBUF = 4
LANES = 16


@functools.lru_cache(maxsize=None)
def _build(n_t: int, n_a: int):
    a_tiles = n_a // GROUP
    f_tiles = D_MODEL // 8

    mesh = plsc.VectorSubcoreMesh(
        core_axis_name="c",
        subcore_axis_name="s",
        num_cores=NUM_CORES,
        num_subcores=NUM_SUBCORES,
    )

    @functools.partial(
        pl.kernel,
        out_type=jax.ShapeDtypeStruct(
            (n_t, f_tiles, a_tiles, 8, GROUP), jnp.float32
        ),
        mesh=mesh,
        scratch_types=[
            pltpu.VMEM((n_t, GROUP), jnp.int32),
            pltpu.VMEM((NBUF, GROUP, D_MODEL), jnp.float32),
            pltpu.VMEM((NBUF, f_tiles, 8, GROUP + 1), jnp.float32),
            pltpu.SemaphoreType.DMA((NBUF,)),
            pltpu.SemaphoreType.DMA((NBUF,)),
        ],
        compiler_params=pltpu.CompilerParams(
            use_tc_tiling_on_sc=False, needs_layout_passes=False
        ),
    )
    def emb_kernel(xt_hbm, table_hbm, out_hbm, idx_v, rows_v, tbuf_v, sem_g, sem_w):
        wid = lax.axis_index("s") * NUM_CORES + lax.axis_index("c")

        pltpu.sync_copy(
            xt_hbm.at[pl.ds(0, n_t), pl.ds(wid * GROUP, GROUP)], idx_v
        )

        def fire_gather(t, buf):
            pltpu.async_copy(
                table_hbm.at[idx_v.at[t]], rows_v.at[buf], sem_g.at[buf]
            )

        def wait_gather(buf):
            pltpu.make_async_copy(
                table_hbm.at[idx_v.at[0]], rows_v.at[buf], sem_g.at[buf]
            ).wait()

        def fire_write(t, buf):
            pltpu.async_copy(
                tbuf_v.at[buf, pl.ds(0, f_tiles), pl.ds(0, 8), pl.ds(0, GROUP)],
                out_hbm.at[t, pl.ds(0, f_tiles), wid],
                sem_w.at[buf],
            )

        def wait_write(buf):
            pltpu.make_async_copy(
                tbuf_v.at[buf, pl.ds(0, f_tiles), pl.ds(0, 8), pl.ds(0, GROUP)],
                out_hbm.at[0, pl.ds(0, f_tiles), 0],
                sem_w.at[buf],
            ).wait()

        for b in range(NBUF):
            fire_gather(b, b)

        base = lax.iota(jnp.int32, LANES)
        fhi = [(base + f0 * LANES) >> 3 for f0 in range(D_MODEL // LANES)]
        flo = [(base + f0 * LANES) & 7 for f0 in range(D_MODEL // LANES)]

        @pl.loop(0, n_t // NBUF)
        def _outer(ti):
            for b in range(NBUF):
                t = ti * NBUF + b
                wait_gather(b)

                @pl.when(t >= NBUF)
                def _():
                    wait_write(b)

                @pl.loop(0, GROUP)
                def _lookup(c):
                    csplat = jnp.full((LANES,), c, jnp.int32)
                    for f0 in range(D_MODEL // LANES):
                        v = rows_v[b, c, pl.ds(f0 * LANES, LANES)]
                        plsc.store_scatter(
                            tbuf_v.at[b], [fhi[f0], flo[f0], csplat], v * 8.0
                        )

                fire_write(t, b)

                @pl.when(t + NBUF < n_t)
                def _():
                    fire_gather(t + NBUF, b)

        for b in range(NBUF):
            wait_write(b)

    return emb_kernel


def kernel(x, table):
    n_a, n_t = x.shape
    xt = jnp.transpose(x).astype(jnp.int32)
    out5 = _build(n_t, n_a)(xt, table)
    return out5.transpose(2, 4, 0, 1, 3).reshape(n_a, n_t, D_MODEL)


# speedup vs baseline: 1.7571x; 1.0077x over previous
import functools

import jax
import jax.numpy as jnp
from jax import lax
from jax.experimental import pallas as pl
from jax.experimental.pallas import tpu as pltpu
from jax.experimental.pallas import tpu_sc as plsc

D_MODEL = 64
NUM_CORES = 2
NUM_SUBCORES = 16
NUM_WORKERS = NUM_CORES * NUM_SUBCORES
GROUP = 128
NBUF = 4
LANES = 16


@functools.lru_cache(maxsize=None)
def _build(n_t: int, n_a: int):
    a_tiles = n_a // GROUP
    f_tiles = D_MODEL // 8

    mesh = plsc.VectorSubcoreMesh(
        core_axis_name="c",
        subcore_axis_name="s",
        num_cores=NUM_CORES,
        num_subcores=NUM_SUBCORES,
    )

    @functools.partial(
        pl.kernel,
        out_type=jax.ShapeDtypeStruct(
            (n_t, f_tiles, a_tiles, 8, GROUP), jnp.float32
        ),
        mesh=mesh,
        scratch_types=[
            pltpu.VMEM((n_t, GROUP), jnp.int32),
            pltpu.VMEM((NBUF, GROUP, D_MODEL), jnp.float32),
            pltpu.VMEM((NBUF, f_tiles, 8, GROUP + 1), jnp.float32),
            pltpu.SemaphoreType.DMA((NBUF,)),
            pltpu.SemaphoreType.DMA((NBUF,)),
        ],
        compiler_params=pltpu.CompilerParams(
            use_tc_tiling_on_sc=False, needs_layout_passes=False
        ),
    )
    def emb_kernel(xt_hbm, table_hbm, out_hbm, idx_v, rows_v, tbuf_v, sem_g, sem_w):
        wid = lax.axis_index("s") * NUM_CORES + lax.axis_index("c")

        pltpu.sync_copy(
            xt_hbm.at[pl.ds(0, n_t), pl.ds(wid * GROUP, GROUP)], idx_v
        )

        def fire_gather(t, buf):
            pltpu.async_copy(
                table_hbm.at[idx_v.at[t]], rows_v.at[buf], sem_g.at[buf]
            )

        def wait_gather(buf):
            pltpu.make_async_copy(
                table_hbm.at[idx_v.at[0]], rows_v.at[buf], sem_g.at[buf]
            ).wait()

        def fire_write(t, buf):
            pltpu.async_copy(
                tbuf_v.at[buf, pl.ds(0, f_tiles), pl.ds(0, 8), pl.ds(0, GROUP)],
                out_hbm.at[t, pl.ds(0, f_tiles), wid],
                sem_w.at[buf],
            )

        def wait_write(buf):
            pltpu.make_async_copy(
                tbuf_v.at[buf, pl.ds(0, f_tiles), pl.ds(0, 8), pl.ds(0, GROUP)],
                out_hbm.at[0, pl.ds(0, f_tiles), 0],
                sem_w.at[buf],
            ).wait()

        for b in range(NBUF):
            fire_gather(b, b)

        base = lax.iota(jnp.int32, LANES)
        fhi = [(base + f0 * LANES) >> 3 for f0 in range(D_MODEL // LANES)]
        flo = [(base + f0 * LANES) & 7 for f0 in range(D_MODEL // LANES)]

        @pl.loop(0, n_t // NBUF)
        def _outer(ti):
            for b in range(NBUF):
                t = ti * NBUF + b
                wait_gather(b)

                @pl.when(t >= NBUF)
                def _():
                    wait_write(b)

                @pl.loop(0, GROUP, unroll=8)
                def _lookup(c):
                    csplat = jnp.full((LANES,), c, jnp.int32)
                    for f0 in range(D_MODEL // LANES):
                        v = rows_v[b, c, pl.ds(f0 * LANES, LANES)]
                        plsc.store_scatter(
                            tbuf_v.at[b], [fhi[f0], flo[f0], csplat], v * 8.0
                        )

                fire_write(t, b)

                @pl.when(t + NBUF < n_t)
                def _():
                    fire_gather(t + NBUF, b)

        for b in range(NBUF):
            wait_write(b)

    return emb_kernel


def kernel(x, table):
    n_a, n_t = x.shape
    xt = jnp.transpose(x).astype(jnp.int32)
    out5 = _build(n_t, n_a)(xt, table)
    return out5.transpose(2, 4, 0, 1, 3).reshape(n_a, n_t, D_MODEL)
